# baseline (speedup 1.0000x reference)
import jax
import jax.numpy as jnp
from jax.experimental import pallas as pl
from jax.experimental.pallas import tpu as pltpu

_PAD_ROWS = 8


def _add_kernel(idx_ref, feats_ref, table_ref, out_ref):
    i = idx_ref[0]
    tbl = table_ref[...]
    rows = jax.lax.broadcasted_iota(jnp.int32, (_PAD_ROWS, 1), 0)
    mask = (rows == i).astype(tbl.dtype)
    row = jnp.sum(tbl * mask, axis=0, keepdims=True)
    out_ref[...] = feats_ref[...] + row


def kernel(feats, table, modality_id):
    B, S, D = feats.shape
    N = B * S
    x = feats.reshape(N, D)
    n_rows = table.shape[0]
    tbl = jnp.pad(table, ((0, _PAD_ROWS - n_rows), (0, 0)))
    idx = jnp.asarray(modality_id, jnp.int32).reshape(1)

    rows_per_block = 1024
    grid = (N // rows_per_block,)

    out = pl.pallas_call(
        _add_kernel,
        grid_spec=pltpu.PrefetchScalarGridSpec(
            num_scalar_prefetch=1,
            grid=grid,
            in_specs=[
                pl.BlockSpec((rows_per_block, D), lambda i, idx_ref: (i, 0)),
                pl.BlockSpec((_PAD_ROWS, D), lambda i, idx_ref: (0, 0)),
            ],
            out_specs=pl.BlockSpec((rows_per_block, D), lambda i, idx_ref: (i, 0)),
        ),
        out_shape=jax.ShapeDtypeStruct((N, D), feats.dtype),
        compiler_params=pltpu.CompilerParams(
            dimension_semantics=("arbitrary",),
        ),
    )(idx, x, tbl)
    return out.reshape(B, S, D)


# speedup vs baseline: 1.0391x; 1.0391x over previous
import jax
import jax.numpy as jnp
from jax.experimental import pallas as pl
from jax.experimental.pallas import tpu as pltpu

_PAD_ROWS = 8


def _add_kernel(idx_ref, feats_ref, table_ref, out_ref):
    i = idx_ref[0]
    tbl = table_ref[...]
    rows = jax.lax.broadcasted_iota(jnp.int32, (_PAD_ROWS, 1), 0)
    mask = (rows == i).astype(tbl.dtype)
    row = jnp.sum(tbl * mask, axis=0, keepdims=True)
    out_ref[...] = feats_ref[...] + row


def kernel(feats, table, modality_id):
    B, S, D = feats.shape
    N = B * S
    x = feats.reshape(N, D)
    n_rows = table.shape[0]
    tbl = jnp.pad(table, ((0, _PAD_ROWS - n_rows), (0, 0)))
    idx = jnp.asarray(modality_id, jnp.int32).reshape(1)

    rows_per_block = 2048
    grid = (N // rows_per_block,)

    out = pl.pallas_call(
        _add_kernel,
        grid_spec=pltpu.PrefetchScalarGridSpec(
            num_scalar_prefetch=1,
            grid=grid,
            in_specs=[
                pl.BlockSpec((rows_per_block, D), lambda i, idx_ref: (i, 0)),
                pl.BlockSpec((_PAD_ROWS, D), lambda i, idx_ref: (0, 0)),
            ],
            out_specs=pl.BlockSpec((rows_per_block, D), lambda i, idx_ref: (i, 0)),
        ),
        out_shape=jax.ShapeDtypeStruct((N, D), feats.dtype),
        compiler_params=pltpu.CompilerParams(
            dimension_semantics=("arbitrary",),
        ),
    )(idx, x, tbl)
    return out.reshape(B, S, D)


# speedup vs baseline: 1.0685x; 1.0283x over previous
import jax
import jax.numpy as jnp
from jax.experimental import pallas as pl
from jax.experimental.pallas import tpu as pltpu

_PAD_ROWS = 8


def _add_kernel(idx_ref, feats_ref, table_ref, out_ref):
    i = idx_ref[0]
    tbl = table_ref[...]
    rows = jax.lax.broadcasted_iota(jnp.int32, (tbl.shape[0], 1), 0)
    mask = (rows == i).astype(tbl.dtype)
    row = jnp.sum(tbl * mask, axis=0, keepdims=True)
    out_ref[...] = feats_ref[...] + row


def kernel(feats, table, modality_id):
    B, S, D = feats.shape
    N = B * S
    x = feats.reshape(N, D)
    n_rows = table.shape[0]
    idx = jnp.asarray(modality_id, jnp.int32).reshape(1)

    rows_per_block = 2048
    grid = (N // rows_per_block,)

    out = pl.pallas_call(
        _add_kernel,
        grid_spec=pltpu.PrefetchScalarGridSpec(
            num_scalar_prefetch=1,
            grid=grid,
            in_specs=[
                pl.BlockSpec((rows_per_block, D), lambda i, idx_ref: (i, 0)),
                pl.BlockSpec((n_rows, D), lambda i, idx_ref: (0, 0)),
            ],
            out_specs=pl.BlockSpec((rows_per_block, D), lambda i, idx_ref: (i, 0)),
        ),
        out_shape=jax.ShapeDtypeStruct((N, D), feats.dtype),
        compiler_params=pltpu.CompilerParams(
            dimension_semantics=("parallel",),
        ),
    )(idx, x, table)
    return out.reshape(B, S, D)


# speedup vs baseline: 1.0729x; 1.0041x over previous
import jax
import jax.numpy as jnp
from jax.experimental import pallas as pl
from jax.experimental.pallas import tpu as pltpu


def _add_kernel(idx_ref, x1_ref, x2_ref, table_ref, out_ref):
    i = idx_ref[0]
    tbl = table_ref[...]
    rows = jax.lax.broadcasted_iota(jnp.int32, (tbl.shape[0], 1), 0)
    mask = (rows == i).astype(tbl.dtype)
    row = jnp.sum(tbl * mask, axis=0, keepdims=True)
    h = x1_ref.shape[0]
    out_ref[:h, :] = x1_ref[...] + row
    out_ref[h:, :] = x2_ref[...] + row


def kernel(feats, table, modality_id):
    B, S, D = feats.shape
    N = B * S
    x = feats.reshape(N, D)
    n_rows = table.shape[0]
    idx = jnp.asarray(modality_id, jnp.int32).reshape(1)

    rows_per_block = 2048
    half = rows_per_block // 2
    grid = (N // rows_per_block,)

    out = pl.pallas_call(
        _add_kernel,
        grid_spec=pltpu.PrefetchScalarGridSpec(
            num_scalar_prefetch=1,
            grid=grid,
            in_specs=[
                pl.BlockSpec((half, D), lambda i, idx_ref: (2 * i, 0)),
                pl.BlockSpec((half, D), lambda i, idx_ref: (2 * i + 1, 0)),
                pl.BlockSpec((n_rows, D), lambda i, idx_ref: (0, 0)),
            ],
            out_specs=pl.BlockSpec((rows_per_block, D), lambda i, idx_ref: (i, 0)),
        ),
        out_shape=jax.ShapeDtypeStruct((N, D), feats.dtype),
        compiler_params=pltpu.CompilerParams(
            dimension_semantics=("parallel",),
        ),
    )(idx, x, x, table)
    return out.reshape(B, S, D)
